# baseline (speedup 1.0000x reference)
import functools
import math

import jax
import jax.numpy as jnp
from jax import lax
from jax.experimental import pallas as pl
from jax.experimental.pallas import tpu as pltpu
from jax.experimental.pallas import tpu_sc as plsc

H = 8
HH = 4
DK = 16
MSGW = HH * DK
ACCW = MSGW + 16

NC = 2
NS = 16
LANES = 16
NB = 2
NPASS = 2


def _proj_body(x_ref, w_ref, b_ref, kv0_ref, kv1_ref, q0_ref, q1_ref):
    xb = x_ref[...]
    y = jnp.dot(xb, w_ref[...], preferred_element_type=jnp.float32) + b_ref[...]
    kv0_ref[...] = y[:, : 2 * MSGW]
    kv1_ref[...] = y[:, 2 * MSGW : 4 * MSGW]
    q0_ref[...] = y[:, 4 * MSGW : 5 * MSGW]
    q1_ref[...] = y[:, 5 * MSGW : 6 * MSGW]


def _run_proj(x, w_all, b_all, bn):
    n, in_dim = x.shape
    grid = n // bn
    wcols = w_all.shape[1]
    return pl.pallas_call(
        _proj_body,
        grid=(grid,),
        in_specs=[
            pl.BlockSpec((bn, in_dim), lambda i: (i, 0)),
            pl.BlockSpec((in_dim, wcols), lambda i: (0, 0)),
            pl.BlockSpec((1, wcols), lambda i: (0, 0)),
        ],
        out_specs=[
            pl.BlockSpec((bn, 2 * MSGW), lambda i: (i, 0)),
            pl.BlockSpec((bn, 2 * MSGW), lambda i: (i, 0)),
            pl.BlockSpec((bn, MSGW), lambda i: (i, 0)),
            pl.BlockSpec((bn, MSGW), lambda i: (i, 0)),
        ],
        out_shape=[
            jax.ShapeDtypeStruct((n, 2 * MSGW), jnp.float32),
            jax.ShapeDtypeStruct((n, 2 * MSGW), jnp.float32),
            jax.ShapeDtypeStruct((n, MSGW), jnp.float32),
            jax.ShapeDtypeStruct((n, MSGW), jnp.float32),
        ],
    )(x, w_all, b_all.reshape(1, -1))


def _sc_edge_kernel(npad, e, ch):
    eps = e // NS
    nchunk = eps // ch
    npc = nchunk // NPASS
    rows_per_sub = npad // NS
    assert eps % ch == 0 and nchunk % NPASS == 0 and npc % NB == 0

    mesh = plsc.VectorSubcoreMesh(core_axis_name="c", subcore_axis_name="s")

    @functools.partial(
        pl.kernel,
        out_type=jax.ShapeDtypeStruct((NC, npad, ACCW), jnp.float32),
        mesh=mesh,
        compiler_params=pltpu.CompilerParams(needs_layout_passes=False,
                                             use_tc_tiling_on_sc=False),
        scratch_types=[
            pltpu.VMEM((nchunk // NPASS, ch), jnp.int32),
            pltpu.VMEM((nchunk // NPASS, ch), jnp.int32),
            pltpu.VMEM((NB, ch, 2 * MSGW), jnp.float32),
            pltpu.VMEM((NB, ch, MSGW), jnp.float32),
            pltpu.VMEM((NB, ch, ACCW), jnp.float32),
            pltpu.VMEM_SHARED((npad, ACCW), jnp.float32),
            pltpu.SemaphoreType.DMA,
            pltpu.SemaphoreType.DMA,
            pltpu.SemaphoreType.DMA,
            pltpu.SemaphoreType.DMA,
        ],
    )
    def edge_kernel(kv0_hbm, kv1_hbm, q0_hbm, q1_hbm, src_hbm, dst_hbm,
                    zero_hbm, out_hbm, sidx, didx, kvb, qb, ob, acc,
                    gsem0, gsem1, ssem0, ssem1):
        c = lax.axis_index("c")
        s = lax.axis_index("s")
        gsem = (gsem0, gsem1)
        ssem = (ssem0, ssem1)

        r0 = s * rows_per_sub
        pltpu.sync_copy(zero_hbm.at[pl.ds(0, rows_per_sub)],
                        acc.at[pl.ds(r0, rows_per_sub)])
        plsc.subcore_barrier()

        lane = lax.broadcasted_iota(jnp.int32, (LANES,), 0)
        x8 = lane ^ 8
        x4 = lane ^ 4
        x2 = lane ^ 2
        x1 = lane ^ 1
        perm = (lane & 1) * 8 + ((lane >> 1) & 1) * 4
        splat_lane = (0, 8, 4, 12)

        def _g(v, ix):
            return jnp.take_along_axis(v, ix, axis=0)

        def fire(b, j):
            @pl.when(c == 0)
            def _():
                pltpu.async_copy(kv0_hbm.at[sidx.at[j]], kvb.at[b], gsem[b])
                pltpu.async_copy(q0_hbm.at[didx.at[j]], qb.at[b], gsem[b])

            @pl.when(c == 1)
            def _():
                pltpu.async_copy(kv1_hbm.at[sidx.at[j]], kvb.at[b], gsem[b])
                pltpu.async_copy(q1_hbm.at[didx.at[j]], qb.at[b], gsem[b])

        def wait_gather(b, j):
            pltpu.make_async_copy(kv0_hbm.at[sidx.at[j]], kvb.at[b],
                                  gsem[b]).wait()
            pltpu.make_async_copy(q0_hbm.at[didx.at[j]], qb.at[b],
                                  gsem[b]).wait()

        def wait_scatter(b, j):
            pltpu.make_async_copy(ob.at[b], acc.at[didx.at[j]],
                                  ssem[b]).wait()

        def compute(b):
            kvb_b = kvb.at[b]
            qb_b = qb.at[b]
            ob_b = ob.at[b]

            @plsc.parallel_loop(0, ch, step=1, unroll=8)
            def _(i):
                m = [qb_b[i, pl.ds(h * DK, DK)] * kvb_b[i, pl.ds(h * DK, DK)]
                     for h in range(HH)]
                a0, a1, a2, a3 = [mm + _g(mm, x8) for mm in m]
                b01 = jnp.where(lane < 8, a0, a1)
                b23 = jnp.where(lane < 8, a2, a3)
                c01 = b01 + _g(b01, x4)
                c23 = b23 + _g(b23, x4)
                dsel = jnp.where((lane & 4) == 0, c01, c23)
                ee = dsel + _g(dsel, x2)
                f = ee + _g(ee, x1)
                p = jnp.exp(f)
                pm = jnp.where(lane < HH, _g(p, perm), 0.0)
                ob_b[i, pl.ds(MSGW, LANES)] = pm
                for h in range(HH):
                    ph = _g(p, jnp.full((LANES,), splat_lane[h], jnp.int32))
                    vh = kvb_b[i, pl.ds(MSGW + h * DK, DK)]
                    ob_b[i, pl.ds(h * DK, DK)] = ph * vh

        def run_pass(pa, carry0):
            pltpu.sync_copy(src_hbm.at[pa, s], sidx)
            pltpu.sync_copy(dst_hbm.at[pa, s], didx)

            for b in range(min(NB, npc)):
                fire(b, b)

            def outer(i, carry):
                j0 = i * NB
                for b in range(NB):
                    j = j0 + b
                    wait_gather(b, j)

                    @pl.when(j >= NB)
                    def _():
                        wait_scatter(b, j - NB)

                    compute(b)
                    pltpu.async_copy(ob.at[b], acc.at[didx.at[j]], ssem[b],
                                     add=True)

                    @pl.when(j + NB < npc)
                    def _():
                        fire(b, j + NB)
                return carry

            lax.fori_loop(0, npc // NB, outer, 0, unroll=False)

            for b in range(NB):
                wait_scatter(b, npc - NB + b)
            return carry0

        lax.fori_loop(0, NPASS, run_pass, 0, unroll=False)

        plsc.subcore_barrier()
        pltpu.sync_copy(acc.at[pl.ds(r0, rows_per_sub)],
                        out_hbm.at[c, pl.ds(r0, rows_per_sub)])

    return edge_kernel


def _final_body(acc_ref, x_ref, wa_ref, ba_ref, beta_ref, out_ref):
    a0 = acc_ref[0]
    a1 = acc_ref[1]
    num = jnp.concatenate([a0[:, :MSGW], a1[:, :MSGW]], axis=1)
    den = jnp.concatenate([a0[:, MSGW : MSGW + HH],
                           a1[:, MSGW : MSGW + HH]], axis=1)
    rows = lax.broadcasted_iota(jnp.int32, (H, H * DK), 0)
    cols = lax.broadcasted_iota(jnp.int32, (H, H * DK), 1) // DK
    emat = (rows == cols).astype(jnp.float32)
    den_full = jnp.dot(den, emat, preferred_element_type=jnp.float32)
    agg = num / (den_full + 1e-9)
    out = jnp.dot(agg, wa_ref[...], preferred_element_type=jnp.float32)
    out_ref[...] = out + ba_ref[...] + beta_ref[0, 0] * x_ref[...]


def _run_final(accs, x, wa_s, ba_s, beta, bn):
    n, in_dim = x.shape
    grid = n // bn
    return pl.pallas_call(
        _final_body,
        grid=(grid,),
        in_specs=[
            pl.BlockSpec((NC, bn, ACCW), lambda i: (0, i, 0)),
            pl.BlockSpec((bn, in_dim), lambda i: (i, 0)),
            pl.BlockSpec((H * DK, in_dim), lambda i: (0, 0)),
            pl.BlockSpec((1, in_dim), lambda i: (0, 0)),
            pl.BlockSpec(memory_space=pltpu.SMEM),
        ],
        out_specs=pl.BlockSpec((bn, in_dim), lambda i: (i, 0)),
        out_shape=jax.ShapeDtypeStruct((n, in_dim), jnp.float32),
    )(accs, x, wa_s, ba_s.reshape(1, -1), beta)


def kernel(x, edge_index, Wk, bk, Wv, bv, Wq, bq, Wa, ba, rel_att, rel_msg,
           rel_pri, skip):
    n, in_dim = x.shape
    e = edge_index.shape[1]

    wk_eff = jnp.einsum("ihd,hdk->ihk", Wk.reshape(in_dim, H, DK),
                        rel_att).reshape(in_dim, H * DK)
    bk_eff = jnp.einsum("hd,hdk->hk", bk.reshape(H, DK),
                        rel_att).reshape(H * DK)
    wv_eff = jnp.einsum("ihd,hdk->ihk", Wv.reshape(in_dim, H, DK),
                        rel_msg).reshape(in_dim, H * DK)
    bv_eff = jnp.einsum("hd,hdk->hk", bv.reshape(H, DK),
                        rel_msg).reshape(H * DK)
    scale = jnp.repeat(rel_pri, DK) / math.sqrt(DK)
    wq_eff = Wq * scale[None, :]
    bq_eff = bq * scale

    w_all = jnp.concatenate(
        [wk_eff[:, :MSGW], wv_eff[:, :MSGW],
         wk_eff[:, MSGW:], wv_eff[:, MSGW:],
         wq_eff[:, :MSGW], wq_eff[:, MSGW:]], axis=1)
    b_all = jnp.concatenate(
        [bk_eff[:MSGW], bv_eff[:MSGW],
         bk_eff[MSGW:], bv_eff[MSGW:],
         bq_eff[:MSGW], bq_eff[MSGW:]], axis=0)

    alpha = jax.nn.sigmoid(skip)
    wa_s = Wa * alpha
    ba_s = ba * alpha
    beta = (1.0 - alpha).astype(jnp.float32).reshape(1, 1)

    kv0, kv1, q0, q1 = _run_proj(x, w_all, b_all, bn=1000)

    ch = 100
    eps = e // NS
    npc = eps // ch // NPASS
    src3 = edge_index[0].reshape(NS, NPASS, npc, ch).transpose(1, 0, 2, 3)
    dst3 = edge_index[1].reshape(NS, NPASS, npc, ch).transpose(1, 0, 2, 3)
    npad = ((n + NS * 8 - 1) // (NS * 8)) * NS * 8
    zero_rows = jnp.zeros((npad // NS, ACCW), jnp.float32)
    accs = _sc_edge_kernel(npad, e, ch=ch)(kv0, kv1, q0, q1, src3, dst3,
                                           zero_rows)

    return _run_final(accs, x, wa_s, ba_s, beta, bn=1000)


# speedup vs baseline: 1.1474x; 1.1474x over previous
import functools
import math

import jax
import jax.numpy as jnp
from jax import lax
from jax.experimental import pallas as pl
from jax.experimental.pallas import tpu as pltpu
from jax.experimental.pallas import tpu_sc as plsc

H = 8
HH = 4
DK = 16
MSGW = HH * DK
ACCW = MSGW + 16

NC = 2
NS = 16
LANES = 16
NB = 2
NPASS = 2


def _proj_body(x_ref, w_ref, b_ref, kv0_ref, kv1_ref, q0_ref, q1_ref):
    xb = x_ref[...]
    y = jnp.dot(xb, w_ref[...], preferred_element_type=jnp.float32) + b_ref[...]
    yb = y.astype(jnp.bfloat16)
    kv0_ref[...] = yb[:, : 2 * MSGW]
    kv1_ref[...] = yb[:, 2 * MSGW : 4 * MSGW]
    q0_ref[...] = yb[:, 4 * MSGW : 5 * MSGW]
    q1_ref[...] = yb[:, 5 * MSGW : 6 * MSGW]


def _run_proj(x, w_all, b_all, bn):
    n, in_dim = x.shape
    grid = n // bn
    wcols = w_all.shape[1]
    return pl.pallas_call(
        _proj_body,
        grid=(grid,),
        in_specs=[
            pl.BlockSpec((bn, in_dim), lambda i: (i, 0)),
            pl.BlockSpec((in_dim, wcols), lambda i: (0, 0)),
            pl.BlockSpec((1, wcols), lambda i: (0, 0)),
        ],
        out_specs=[
            pl.BlockSpec((bn, 2 * MSGW), lambda i: (i, 0)),
            pl.BlockSpec((bn, 2 * MSGW), lambda i: (i, 0)),
            pl.BlockSpec((bn, MSGW), lambda i: (i, 0)),
            pl.BlockSpec((bn, MSGW), lambda i: (i, 0)),
        ],
        out_shape=[
            jax.ShapeDtypeStruct((n, 2 * MSGW), jnp.bfloat16),
            jax.ShapeDtypeStruct((n, 2 * MSGW), jnp.bfloat16),
            jax.ShapeDtypeStruct((n, MSGW), jnp.bfloat16),
            jax.ShapeDtypeStruct((n, MSGW), jnp.bfloat16),
        ],
    )(x, w_all, b_all.reshape(1, -1))


def _sc_edge_kernel(npad, e, ch):
    eps = e // NS
    nchunk = eps // ch
    npc = nchunk // NPASS
    rows_per_sub = npad // NS
    assert eps % ch == 0 and nchunk % NPASS == 0 and npc % NB == 0

    mesh = plsc.VectorSubcoreMesh(core_axis_name="c", subcore_axis_name="s")

    @functools.partial(
        pl.kernel,
        out_type=jax.ShapeDtypeStruct((NC, npad, ACCW), jnp.float32),
        mesh=mesh,
        compiler_params=pltpu.CompilerParams(needs_layout_passes=False,
                                             use_tc_tiling_on_sc=False),
        scratch_types=[
            pltpu.VMEM((nchunk // NPASS, ch), jnp.int32),
            pltpu.VMEM((nchunk // NPASS, ch), jnp.int32),
            pltpu.VMEM((NB, ch, 2 * MSGW), jnp.bfloat16),
            pltpu.VMEM((NB, ch, MSGW), jnp.bfloat16),
            pltpu.VMEM((NB, ch, ACCW), jnp.float32),
            pltpu.VMEM_SHARED((npad, ACCW), jnp.float32),
            pltpu.SemaphoreType.DMA,
            pltpu.SemaphoreType.DMA,
            pltpu.SemaphoreType.DMA,
            pltpu.SemaphoreType.DMA,
        ],
    )
    def edge_kernel(kv0_hbm, kv1_hbm, q0_hbm, q1_hbm, src_hbm, dst_hbm,
                    zero_hbm, out_hbm, sidx, didx, kvb, qb, ob, acc,
                    gsem0, gsem1, ssem0, ssem1):
        c = lax.axis_index("c")
        s = lax.axis_index("s")
        gsem = (gsem0, gsem1)
        ssem = (ssem0, ssem1)

        r0 = s * rows_per_sub
        pltpu.sync_copy(zero_hbm.at[pl.ds(0, rows_per_sub)],
                        acc.at[pl.ds(r0, rows_per_sub)])
        plsc.subcore_barrier()

        lane = lax.broadcasted_iota(jnp.int32, (LANES,), 0)
        x4 = lane ^ 4
        x2 = lane ^ 2
        x1 = lane ^ 1
        pidx = (lane & 1) * 8

        def _g(v, ix):
            return jnp.take_along_axis(v, ix, axis=0)

        def fire(b, j):
            @pl.when(c == 0)
            def _():
                pltpu.async_copy(kv0_hbm.at[sidx.at[j]], kvb.at[b], gsem[b])
                pltpu.async_copy(q0_hbm.at[didx.at[j]], qb.at[b], gsem[b])

            @pl.when(c == 1)
            def _():
                pltpu.async_copy(kv1_hbm.at[sidx.at[j]], kvb.at[b], gsem[b])
                pltpu.async_copy(q1_hbm.at[didx.at[j]], qb.at[b], gsem[b])

        def wait_gather(b, j):
            pltpu.make_async_copy(kv0_hbm.at[sidx.at[j]], kvb.at[b],
                                  gsem[b]).wait()
            pltpu.make_async_copy(q0_hbm.at[didx.at[j]], qb.at[b],
                                  gsem[b]).wait()

        def wait_scatter(b, j):
            pltpu.make_async_copy(ob.at[b], acc.at[didx.at[j]],
                                  ssem[b]).wait()

        def compute(b):
            kvb_b = kvb.at[b]
            qb_b = qb.at[b]
            ob_b = ob.at[b]

            unp = functools.partial(plsc.unpack,
                                    format=plsc.PackFormat.INTERLEAVED)

            @plsc.parallel_loop(0, ch, step=1, unroll=4)
            def _(i):
                ps = []
                for pr in range(HH // 2):
                    qe, qo = unp(qb_b[i, pl.ds(pr * 32, 32)])
                    ke, ko = unp(kvb_b[i, pl.ds(pr * 32, 32)])
                    m = qe * ke + qo * ko
                    r = m + _g(m, x4)
                    r = r + _g(r, x2)
                    r = r + _g(r, x1)
                    ps.append(jnp.exp(r))
                p01, p23 = ps
                pden = jnp.where((lane & 2) == 0, _g(p01, pidx),
                                 _g(p23, pidx))
                ob_b[i, pl.ds(MSGW, LANES)] = jnp.where(lane < HH, pden, 0.0)
                for pr in range(HH // 2):
                    ve, vo = unp(kvb_b[i, pl.ds(MSGW + pr * 32, 32)])
                    ob_b[i, pl.ds(pr * 32, DK)] = ps[pr] * ve
                    ob_b[i, pl.ds(pr * 32 + DK, DK)] = ps[pr] * vo

        def run_pass(pa, carry0):
            pltpu.sync_copy(src_hbm.at[pa, s], sidx)
            pltpu.sync_copy(dst_hbm.at[pa, s], didx)

            for b in range(min(NB, npc)):
                fire(b, b)

            def outer(i, carry):
                j0 = i * NB
                for b in range(NB):
                    j = j0 + b
                    wait_gather(b, j)

                    @pl.when(j >= NB)
                    def _():
                        wait_scatter(b, j - NB)

                    compute(b)
                    pltpu.async_copy(ob.at[b], acc.at[didx.at[j]], ssem[b],
                                     add=True)

                    @pl.when(j + NB < npc)
                    def _():
                        fire(b, j + NB)
                return carry

            lax.fori_loop(0, npc // NB, outer, 0, unroll=False)

            for b in range(NB):
                wait_scatter(b, npc - NB + b)
            return carry0

        lax.fori_loop(0, NPASS, run_pass, 0, unroll=False)

        plsc.subcore_barrier()
        pltpu.sync_copy(acc.at[pl.ds(r0, rows_per_sub)],
                        out_hbm.at[c, pl.ds(r0, rows_per_sub)])

    return edge_kernel


def _final_body(acc_ref, x_ref, wa_ref, ba_ref, beta_ref, out_ref):
    a0 = acc_ref[0]
    a1 = acc_ref[1]
    num = jnp.concatenate([a0[:, :MSGW], a1[:, :MSGW]], axis=1)
    den = jnp.concatenate([a0[:, MSGW : MSGW + HH],
                           a1[:, MSGW : MSGW + HH]], axis=1)
    rows = lax.broadcasted_iota(jnp.int32, (H, H * DK), 0)
    colv = lax.broadcasted_iota(jnp.int32, (H, H * DK), 1)
    head_col = (4 * (colv // 64) + 2 * ((colv % 64) // 32)
                + ((colv // 8) & 1))
    emat = (rows == head_col).astype(jnp.float32)
    den_full = jnp.dot(den, emat, preferred_element_type=jnp.float32)
    agg = num / (den_full + 1e-9)
    out = jnp.dot(agg, wa_ref[...], preferred_element_type=jnp.float32)
    out_ref[...] = out + ba_ref[...] + beta_ref[0, 0] * x_ref[...]


def _run_final(accs, x, wa_s, ba_s, beta, bn):
    n, in_dim = x.shape
    grid = n // bn
    return pl.pallas_call(
        _final_body,
        grid=(grid,),
        in_specs=[
            pl.BlockSpec((NC, bn, ACCW), lambda i: (0, i, 0)),
            pl.BlockSpec((bn, in_dim), lambda i: (i, 0)),
            pl.BlockSpec((H * DK, in_dim), lambda i: (0, 0)),
            pl.BlockSpec((1, in_dim), lambda i: (0, 0)),
            pl.BlockSpec(memory_space=pltpu.SMEM),
        ],
        out_specs=pl.BlockSpec((bn, in_dim), lambda i: (i, 0)),
        out_shape=jax.ShapeDtypeStruct((n, in_dim), jnp.float32),
    )(accs, x, wa_s, ba_s.reshape(1, -1), beta)


def kernel(x, edge_index, Wk, bk, Wv, bv, Wq, bq, Wa, ba, rel_att, rel_msg,
           rel_pri, skip):
    n, in_dim = x.shape
    e = edge_index.shape[1]

    wk_eff = jnp.einsum("ihd,hdk->ihk", Wk.reshape(in_dim, H, DK),
                        rel_att).reshape(in_dim, H * DK)
    bk_eff = jnp.einsum("hd,hdk->hk", bk.reshape(H, DK),
                        rel_att).reshape(H * DK)
    wv_eff = jnp.einsum("ihd,hdk->ihk", Wv.reshape(in_dim, H, DK),
                        rel_msg).reshape(in_dim, H * DK)
    bv_eff = jnp.einsum("hd,hdk->hk", bv.reshape(H, DK),
                        rel_msg).reshape(H * DK)
    scale = jnp.repeat(rel_pri, DK) / math.sqrt(DK)
    wq_eff = Wq * scale[None, :]
    bq_eff = bq * scale

    w_all = jnp.concatenate(
        [wk_eff[:, :MSGW], wv_eff[:, :MSGW],
         wk_eff[:, MSGW:], wv_eff[:, MSGW:],
         wq_eff[:, :MSGW], wq_eff[:, MSGW:]], axis=1)
    b_all = jnp.concatenate(
        [bk_eff[:MSGW], bv_eff[:MSGW],
         bk_eff[MSGW:], bv_eff[MSGW:],
         bq_eff[:MSGW], bq_eff[MSGW:]], axis=0)

    alpha = jax.nn.sigmoid(skip)
    col = jnp.arange(H * DK)
    head = 4 * (col // 64) + 2 * ((col % 64) // 32) + ((col // 8) & 1)
    dd = 2 * (col % 8) + ((col // 16) & 1)
    wa_s = (Wa * alpha)[head * DK + dd, :]
    ba_s = ba * alpha
    beta = (1.0 - alpha).astype(jnp.float32).reshape(1, 1)

    kv0, kv1, q0, q1 = _run_proj(x, w_all, b_all, bn=1000)

    ch = 100
    eps = e // NS
    npc = eps // ch // NPASS
    src3 = edge_index[0].reshape(NS, NPASS, npc, ch).transpose(1, 0, 2, 3)
    dst3 = edge_index[1].reshape(NS, NPASS, npc, ch).transpose(1, 0, 2, 3)
    npad = ((n + NS * 8 - 1) // (NS * 8)) * NS * 8
    zero_rows = jnp.zeros((npad // NS, ACCW), jnp.float32)
    accs = _sc_edge_kernel(npad, e, ch=ch)(kv0, kv1, q0, q1, src3, dst3,
                                           zero_rows)

    return _run_final(accs, x, wa_s, ba_s, beta, bn=1000)


# speedup vs baseline: 1.1485x; 1.0010x over previous
import functools
import math

import jax
import jax.numpy as jnp
from jax import lax
from jax.experimental import pallas as pl
from jax.experimental.pallas import tpu as pltpu
from jax.experimental.pallas import tpu_sc as plsc

H = 8
HH = 4
DK = 16
MSGW = HH * DK
ACCW = MSGW + 16

NC = 2
NS = 16
LANES = 16
NB = 2
NPASS = 2


def _proj_body(x_ref, w_ref, b_ref, kv0_ref, kv1_ref, q0_ref, q1_ref):
    xb = x_ref[...]
    y = jnp.dot(xb, w_ref[...], preferred_element_type=jnp.float32) + b_ref[...]
    yb = y.astype(jnp.bfloat16)
    kv0_ref[...] = yb[:, : 2 * MSGW]
    kv1_ref[...] = yb[:, 2 * MSGW : 4 * MSGW]
    q0_ref[...] = yb[:, 4 * MSGW : 5 * MSGW]
    q1_ref[...] = yb[:, 5 * MSGW : 6 * MSGW]


def _run_proj(x, w_all, b_all, bn):
    n, in_dim = x.shape
    grid = n // bn
    wcols = w_all.shape[1]
    return pl.pallas_call(
        _proj_body,
        grid=(grid,),
        in_specs=[
            pl.BlockSpec((bn, in_dim), lambda i: (i, 0)),
            pl.BlockSpec((in_dim, wcols), lambda i: (0, 0)),
            pl.BlockSpec((1, wcols), lambda i: (0, 0)),
        ],
        out_specs=[
            pl.BlockSpec((bn, 2 * MSGW), lambda i: (i, 0)),
            pl.BlockSpec((bn, 2 * MSGW), lambda i: (i, 0)),
            pl.BlockSpec((bn, MSGW), lambda i: (i, 0)),
            pl.BlockSpec((bn, MSGW), lambda i: (i, 0)),
        ],
        out_shape=[
            jax.ShapeDtypeStruct((n, 2 * MSGW), jnp.bfloat16),
            jax.ShapeDtypeStruct((n, 2 * MSGW), jnp.bfloat16),
            jax.ShapeDtypeStruct((n, MSGW), jnp.bfloat16),
            jax.ShapeDtypeStruct((n, MSGW), jnp.bfloat16),
        ],
    )(x, w_all, b_all.reshape(1, -1))


def _sc_edge_kernel(npad, e, ch):
    eps = e // NS
    nchunk = eps // ch
    npc = nchunk // NPASS
    rows_per_sub = npad // NS
    assert eps % ch == 0 and nchunk % NPASS == 0 and npc % NB == 0

    mesh = plsc.VectorSubcoreMesh(core_axis_name="c", subcore_axis_name="s")

    @functools.partial(
        pl.kernel,
        out_type=jax.ShapeDtypeStruct((NC, npad, ACCW), jnp.float32),
        mesh=mesh,
        compiler_params=pltpu.CompilerParams(needs_layout_passes=False,
                                             use_tc_tiling_on_sc=False),
        scratch_types=[
            pltpu.VMEM((nchunk // NPASS, ch), jnp.int32),
            pltpu.VMEM((nchunk // NPASS, ch), jnp.int32),
            pltpu.VMEM((NB, ch, 2 * MSGW), jnp.bfloat16),
            pltpu.VMEM((NB, ch, MSGW), jnp.bfloat16),
            pltpu.VMEM((NB, ch, ACCW), jnp.float32),
            pltpu.VMEM_SHARED((npad, ACCW), jnp.float32),
            pltpu.SemaphoreType.DMA,
            pltpu.SemaphoreType.DMA,
            pltpu.SemaphoreType.DMA,
            pltpu.SemaphoreType.DMA,
        ],
    )
    def edge_kernel(kv0_hbm, kv1_hbm, q0_hbm, q1_hbm, src_hbm, dst_hbm,
                    zero_hbm, out_hbm, sidx, didx, kvb, qb, ob, acc,
                    gsem0, gsem1, ssem0, ssem1):
        c = lax.axis_index("c")
        s = lax.axis_index("s")
        gsem = (gsem0, gsem1)
        ssem = (ssem0, ssem1)

        r0 = s * rows_per_sub
        pltpu.sync_copy(zero_hbm.at[pl.ds(0, rows_per_sub)],
                        acc.at[pl.ds(r0, rows_per_sub)])
        plsc.subcore_barrier()

        lane = lax.broadcasted_iota(jnp.int32, (LANES,), 0)
        x4 = lane ^ 4
        x2 = lane ^ 2
        x1 = lane ^ 1
        pidx = (lane & 1) * 8

        def _g(v, ix):
            return jnp.take_along_axis(v, ix, axis=0)

        def fire(b, j):
            @pl.when(c == 0)
            def _():
                pltpu.async_copy(kv0_hbm.at[sidx.at[j]], kvb.at[b], gsem[b])
                pltpu.async_copy(q0_hbm.at[didx.at[j]], qb.at[b], gsem[b])

            @pl.when(c == 1)
            def _():
                pltpu.async_copy(kv1_hbm.at[sidx.at[j]], kvb.at[b], gsem[b])
                pltpu.async_copy(q1_hbm.at[didx.at[j]], qb.at[b], gsem[b])

        def wait_gather(b, j):
            pltpu.make_async_copy(kv0_hbm.at[sidx.at[j]], kvb.at[b],
                                  gsem[b]).wait()
            pltpu.make_async_copy(q0_hbm.at[didx.at[j]], qb.at[b],
                                  gsem[b]).wait()

        def wait_scatter(b, j):
            pltpu.make_async_copy(ob.at[b], acc.at[didx.at[j]],
                                  ssem[b]).wait()

        def compute(b):
            kvb_b = kvb.at[b]
            qb_b = qb.at[b]
            ob_b = ob.at[b]

            unp = functools.partial(plsc.unpack,
                                    format=plsc.PackFormat.INTERLEAVED)

            @plsc.parallel_loop(0, ch, step=1, unroll=4)
            def _(i):
                ps = []
                for pr in range(HH // 2):
                    qe, qo = unp(qb_b[i, pl.ds(pr * 32, 32)])
                    ke, ko = unp(kvb_b[i, pl.ds(pr * 32, 32)])
                    m = qe * ke + qo * ko
                    r = m + _g(m, x4)
                    r = r + _g(r, x2)
                    r = r + _g(r, x1)
                    ps.append(jnp.exp(r))
                p01, p23 = ps
                pden = jnp.where((lane & 2) == 0, _g(p01, pidx),
                                 _g(p23, pidx))
                ob_b[i, pl.ds(MSGW, LANES)] = jnp.where(lane < HH, pden, 0.0)
                for pr in range(HH // 2):
                    ve, vo = unp(kvb_b[i, pl.ds(MSGW + pr * 32, 32)])
                    ob_b[i, pl.ds(pr * 32, DK)] = ps[pr] * ve
                    ob_b[i, pl.ds(pr * 32 + DK, DK)] = ps[pr] * vo

        def run_pass(pa, carry0):
            pltpu.sync_copy(src_hbm.at[pa, s], sidx)
            pltpu.sync_copy(dst_hbm.at[pa, s], didx)

            for b in range(min(NB, npc)):
                fire(b, b)

            def outer(i, carry):
                j0 = i * NB
                for b in range(NB):
                    j = j0 + b
                    wait_gather(b, j)

                    @pl.when(j >= NB)
                    def _():
                        wait_scatter(b, j - NB)

                    compute(b)
                    pltpu.async_copy(ob.at[b], acc.at[didx.at[j]], ssem[b],
                                     add=True)

                    @pl.when(j + NB < npc)
                    def _():
                        fire(b, j + NB)
                return carry

            lax.fori_loop(0, npc // NB, outer, 0, unroll=False)

            for b in range(NB):
                wait_scatter(b, npc - NB + b)
            return carry0

        lax.fori_loop(0, NPASS, run_pass, 0, unroll=False)

        plsc.subcore_barrier()
        pltpu.sync_copy(acc.at[pl.ds(r0, rows_per_sub)],
                        out_hbm.at[c, pl.ds(r0, rows_per_sub)])

    return edge_kernel


def _final_body(acc_ref, x_ref, wa_ref, ba_ref, beta_ref, out_ref):
    a0 = acc_ref[0]
    a1 = acc_ref[1]
    num = jnp.concatenate([a0[:, :MSGW], a1[:, :MSGW]], axis=1)
    den = jnp.concatenate([a0[:, MSGW : MSGW + HH],
                           a1[:, MSGW : MSGW + HH]], axis=1)
    rows = lax.broadcasted_iota(jnp.int32, (H, H * DK), 0)
    colv = lax.broadcasted_iota(jnp.int32, (H, H * DK), 1)
    head_col = (4 * (colv // 64) + 2 * ((colv % 64) // 32)
                + ((colv // 8) & 1))
    emat = (rows == head_col).astype(jnp.float32)
    den_full = jnp.dot(den, emat, preferred_element_type=jnp.float32)
    agg = num / (den_full + 1e-9)
    out = jnp.dot(agg, wa_ref[...], preferred_element_type=jnp.float32)
    out_ref[...] = out + ba_ref[...] + beta_ref[0, 0] * x_ref[...]


def _run_final(accs, x, wa_s, ba_s, beta, bn):
    n, in_dim = x.shape
    grid = n // bn
    return pl.pallas_call(
        _final_body,
        grid=(grid,),
        in_specs=[
            pl.BlockSpec((NC, bn, ACCW), lambda i: (0, i, 0)),
            pl.BlockSpec((bn, in_dim), lambda i: (i, 0)),
            pl.BlockSpec((H * DK, in_dim), lambda i: (0, 0)),
            pl.BlockSpec((1, in_dim), lambda i: (0, 0)),
            pl.BlockSpec(memory_space=pltpu.SMEM),
        ],
        out_specs=pl.BlockSpec((bn, in_dim), lambda i: (i, 0)),
        out_shape=jax.ShapeDtypeStruct((n, in_dim), jnp.float32),
    )(accs, x, wa_s, ba_s.reshape(1, -1), beta)


def kernel(x, edge_index, Wk, bk, Wv, bv, Wq, bq, Wa, ba, rel_att, rel_msg,
           rel_pri, skip):
    n, in_dim = x.shape
    e = edge_index.shape[1]

    wk_eff = jnp.einsum("ihd,hdk->ihk", Wk.reshape(in_dim, H, DK),
                        rel_att).reshape(in_dim, H * DK)
    bk_eff = jnp.einsum("hd,hdk->hk", bk.reshape(H, DK),
                        rel_att).reshape(H * DK)
    wv_eff = jnp.einsum("ihd,hdk->ihk", Wv.reshape(in_dim, H, DK),
                        rel_msg).reshape(in_dim, H * DK)
    bv_eff = jnp.einsum("hd,hdk->hk", bv.reshape(H, DK),
                        rel_msg).reshape(H * DK)
    scale = jnp.repeat(rel_pri, DK) / math.sqrt(DK)
    wq_eff = Wq * scale[None, :]
    bq_eff = bq * scale

    w_all = jnp.concatenate(
        [wk_eff[:, :MSGW], wv_eff[:, :MSGW],
         wk_eff[:, MSGW:], wv_eff[:, MSGW:],
         wq_eff[:, :MSGW], wq_eff[:, MSGW:]], axis=1)
    b_all = jnp.concatenate(
        [bk_eff[:MSGW], bv_eff[:MSGW],
         bk_eff[MSGW:], bv_eff[MSGW:],
         bq_eff[:MSGW], bq_eff[MSGW:]], axis=0)

    alpha = jax.nn.sigmoid(skip)
    col = jnp.arange(H * DK)
    head = 4 * (col // 64) + 2 * ((col % 64) // 32) + ((col // 8) & 1)
    dd = 2 * (col % 8) + ((col // 16) & 1)
    wa_s = (Wa * alpha)[head * DK + dd, :]
    ba_s = ba * alpha
    beta = (1.0 - alpha).astype(jnp.float32).reshape(1, 1)

    kv0, kv1, q0, q1 = _run_proj(x, w_all, b_all, bn=1000)

    ch = 100
    eps = e // NS
    npc = eps // ch // NPASS
    src3 = edge_index[0].reshape(NPASS, NS, npc, ch)
    dst3 = edge_index[1].reshape(NPASS, NS, npc, ch)
    npad = ((n + NS * 8 - 1) // (NS * 8)) * NS * 8
    zero_rows = jnp.zeros((npad // NS, ACCW), jnp.float32)
    accs = _sc_edge_kernel(npad, e, ch=ch)(kv0, kv1, q0, q1, src3, dst3,
                                           zero_rows)

    return _run_final(accs, x, wa_s, ba_s, beta, bn=1000)


# speedup vs baseline: 1.1843x; 1.0312x over previous
import functools
import math

import jax
import jax.numpy as jnp
from jax import lax
from jax.experimental import pallas as pl
from jax.experimental.pallas import tpu as pltpu
from jax.experimental.pallas import tpu_sc as plsc

H = 8
HH = 4
DK = 16
MSGW = HH * DK
ACCW = MSGW + 16

NC = 2
NS = 16
LANES = 16
NB = 2
NPASS = 2


def _proj_body(x_ref, w_ref, b_ref, kv0_ref, kv1_ref, q0_ref, q1_ref):
    xb = x_ref[...]
    y = jnp.dot(xb, w_ref[...], preferred_element_type=jnp.float32) + b_ref[...]
    yb = y.astype(jnp.bfloat16)
    kv0_ref[...] = yb[:, : 2 * MSGW]
    kv1_ref[...] = yb[:, 2 * MSGW : 4 * MSGW]
    q0_ref[...] = yb[:, 4 * MSGW : 5 * MSGW]
    q1_ref[...] = yb[:, 5 * MSGW : 6 * MSGW]


def _run_proj(x, w_all, b_all, bn):
    n, in_dim = x.shape
    grid = n // bn
    wcols = w_all.shape[1]
    return pl.pallas_call(
        _proj_body,
        grid=(grid,),
        in_specs=[
            pl.BlockSpec((bn, in_dim), lambda i: (i, 0)),
            pl.BlockSpec((in_dim, wcols), lambda i: (0, 0)),
            pl.BlockSpec((1, wcols), lambda i: (0, 0)),
        ],
        out_specs=[
            pl.BlockSpec((bn, 2 * MSGW), lambda i: (i, 0)),
            pl.BlockSpec((bn, 2 * MSGW), lambda i: (i, 0)),
            pl.BlockSpec((bn, MSGW), lambda i: (i, 0)),
            pl.BlockSpec((bn, MSGW), lambda i: (i, 0)),
        ],
        out_shape=[
            jax.ShapeDtypeStruct((n, 2 * MSGW), jnp.bfloat16),
            jax.ShapeDtypeStruct((n, 2 * MSGW), jnp.bfloat16),
            jax.ShapeDtypeStruct((n, MSGW), jnp.bfloat16),
            jax.ShapeDtypeStruct((n, MSGW), jnp.bfloat16),
        ],
    )(x, w_all, b_all.reshape(1, -1))


def _sc_edge_kernel(npad, e, ch):
    eps = e // NS
    nchunk = eps // ch
    npc = nchunk // NPASS
    rows_per_sub = npad // NS
    assert eps % ch == 0 and nchunk % NPASS == 0 and npc % NB == 0

    mesh = plsc.VectorSubcoreMesh(core_axis_name="c", subcore_axis_name="s")

    @functools.partial(
        pl.kernel,
        out_type=jax.ShapeDtypeStruct((NC, npad, ACCW), jnp.float32),
        mesh=mesh,
        compiler_params=pltpu.CompilerParams(needs_layout_passes=False,
                                             use_tc_tiling_on_sc=False),
        scratch_types=[
            pltpu.VMEM((nchunk // NPASS, ch), jnp.int32),
            pltpu.VMEM((nchunk // NPASS, ch), jnp.int32),
            pltpu.VMEM((NB, ch, 2 * MSGW), jnp.bfloat16),
            pltpu.VMEM((NB, ch, MSGW), jnp.bfloat16),
            pltpu.VMEM((NB, ch, ACCW), jnp.float32),
            pltpu.VMEM_SHARED((npad, ACCW), jnp.float32),
            pltpu.SemaphoreType.DMA,
            pltpu.SemaphoreType.DMA,
            pltpu.SemaphoreType.DMA,
            pltpu.SemaphoreType.DMA,
        ],
    )
    def edge_kernel(kv0_hbm, kv1_hbm, q0_hbm, q1_hbm, src_hbm, dst_hbm,
                    zero_hbm, out_hbm, sidx, didx, kvb, qb, ob, acc,
                    gsem0, gsem1, ssem0, ssem1):
        c = lax.axis_index("c")
        s = lax.axis_index("s")
        gsem = (gsem0, gsem1)
        ssem = (ssem0, ssem1)

        r0 = s * rows_per_sub
        pltpu.sync_copy(zero_hbm.at[pl.ds(0, rows_per_sub)],
                        acc.at[pl.ds(r0, rows_per_sub)])
        plsc.subcore_barrier()

        lane = lax.broadcasted_iota(jnp.int32, (LANES,), 0)
        x4 = lane ^ 4
        x2 = lane ^ 2
        x1 = lane ^ 1
        pidx = (lane & 1) * 8

        def _g(v, ix):
            return jnp.take_along_axis(v, ix, axis=0)

        def fire(b, j):
            @pl.when(c == 0)
            def _():
                pltpu.async_copy(kv0_hbm.at[sidx.at[j]], kvb.at[b], gsem[b])
                pltpu.async_copy(q0_hbm.at[didx.at[j]], qb.at[b], gsem[b])

            @pl.when(c == 1)
            def _():
                pltpu.async_copy(kv1_hbm.at[sidx.at[j]], kvb.at[b], gsem[b])
                pltpu.async_copy(q1_hbm.at[didx.at[j]], qb.at[b], gsem[b])

        def wait_gather(b, j):
            pltpu.make_async_copy(kv0_hbm.at[sidx.at[j]], kvb.at[b],
                                  gsem[b]).wait()
            pltpu.make_async_copy(q0_hbm.at[didx.at[j]], qb.at[b],
                                  gsem[b]).wait()

        def wait_scatter(b, j):
            pltpu.make_async_copy(ob.at[b], acc.at[didx.at[j]],
                                  ssem[b]).wait()

        def compute(b):
            kvb_b = kvb.at[b]
            qb_b = qb.at[b]
            ob_b = ob.at[b]

            unp = functools.partial(plsc.unpack,
                                    format=plsc.PackFormat.INTERLEAVED)

            @plsc.parallel_loop(0, ch, step=1, unroll=4)
            def _(i):
                ps = []
                for pr in range(HH // 2):
                    mq = (qb_b[i, pl.ds(pr * 32, 32)]
                          * kvb_b[i, pl.ds(pr * 32, 32)])
                    me, mo = unp(mq)
                    m = me + mo
                    r = m + _g(m, x4)
                    r = r + _g(r, x2)
                    r = r + _g(r, x1)
                    ps.append(jnp.exp(r))
                p01, p23 = ps
                pden = jnp.where((lane & 2) == 0, _g(p01, pidx),
                                 _g(p23, pidx))
                ob_b[i, pl.ds(MSGW, LANES)] = jnp.where(lane < HH, pden, 0.0)
                for pr in range(HH // 2):
                    ve, vo = unp(kvb_b[i, pl.ds(MSGW + pr * 32, 32)])
                    ob_b[i, pl.ds(pr * 32, DK)] = ps[pr] * ve
                    ob_b[i, pl.ds(pr * 32 + DK, DK)] = ps[pr] * vo

        def run_pass(pa, carry0):
            pltpu.sync_copy(src_hbm.at[pa, s], sidx)
            pltpu.sync_copy(dst_hbm.at[pa, s], didx)

            for b in range(min(NB, npc)):
                fire(b, b)

            def outer(i, carry):
                j0 = i * NB
                for b in range(NB):
                    j = j0 + b
                    wait_gather(b, j)

                    @pl.when(j >= NB)
                    def _():
                        wait_scatter(b, j - NB)

                    compute(b)
                    pltpu.async_copy(ob.at[b], acc.at[didx.at[j]], ssem[b],
                                     add=True)

                    @pl.when(j + NB < npc)
                    def _():
                        fire(b, j + NB)
                return carry

            lax.fori_loop(0, npc // NB, outer, 0, unroll=False)

            for b in range(NB):
                wait_scatter(b, npc - NB + b)
            return carry0

        lax.fori_loop(0, NPASS, run_pass, 0, unroll=False)

        plsc.subcore_barrier()
        pltpu.sync_copy(acc.at[pl.ds(r0, rows_per_sub)],
                        out_hbm.at[c, pl.ds(r0, rows_per_sub)])

    return edge_kernel


def _final_body(acc_ref, x_ref, wa_ref, ba_ref, beta_ref, out_ref):
    a0 = acc_ref[0]
    a1 = acc_ref[1]
    num = jnp.concatenate([a0[:, :MSGW], a1[:, :MSGW]], axis=1)
    den = jnp.concatenate([a0[:, MSGW : MSGW + HH],
                           a1[:, MSGW : MSGW + HH]], axis=1)
    rows = lax.broadcasted_iota(jnp.int32, (H, H * DK), 0)
    colv = lax.broadcasted_iota(jnp.int32, (H, H * DK), 1)
    head_col = (4 * (colv // 64) + 2 * ((colv % 64) // 32)
                + ((colv // 8) & 1))
    emat = (rows == head_col).astype(jnp.float32)
    den_full = jnp.dot(den, emat, preferred_element_type=jnp.float32)
    agg = num / (den_full + 1e-9)
    out = jnp.dot(agg, wa_ref[...], preferred_element_type=jnp.float32)
    out_ref[...] = out + ba_ref[...] + beta_ref[0, 0] * x_ref[...]


def _run_final(accs, x, wa_s, ba_s, beta, bn):
    n, in_dim = x.shape
    grid = n // bn
    return pl.pallas_call(
        _final_body,
        grid=(grid,),
        in_specs=[
            pl.BlockSpec((NC, bn, ACCW), lambda i: (0, i, 0)),
            pl.BlockSpec((bn, in_dim), lambda i: (i, 0)),
            pl.BlockSpec((H * DK, in_dim), lambda i: (0, 0)),
            pl.BlockSpec((1, in_dim), lambda i: (0, 0)),
            pl.BlockSpec(memory_space=pltpu.SMEM),
        ],
        out_specs=pl.BlockSpec((bn, in_dim), lambda i: (i, 0)),
        out_shape=jax.ShapeDtypeStruct((n, in_dim), jnp.float32),
    )(accs, x, wa_s, ba_s.reshape(1, -1), beta)


def kernel(x, edge_index, Wk, bk, Wv, bv, Wq, bq, Wa, ba, rel_att, rel_msg,
           rel_pri, skip):
    n, in_dim = x.shape
    e = edge_index.shape[1]

    wk_eff = jnp.einsum("ihd,hdk->ihk", Wk.reshape(in_dim, H, DK),
                        rel_att).reshape(in_dim, H * DK)
    bk_eff = jnp.einsum("hd,hdk->hk", bk.reshape(H, DK),
                        rel_att).reshape(H * DK)
    wv_eff = jnp.einsum("ihd,hdk->ihk", Wv.reshape(in_dim, H, DK),
                        rel_msg).reshape(in_dim, H * DK)
    bv_eff = jnp.einsum("hd,hdk->hk", bv.reshape(H, DK),
                        rel_msg).reshape(H * DK)
    scale = jnp.repeat(rel_pri, DK) / math.sqrt(DK)
    wq_eff = Wq * scale[None, :]
    bq_eff = bq * scale

    w_all = jnp.concatenate(
        [wk_eff[:, :MSGW], wv_eff[:, :MSGW],
         wk_eff[:, MSGW:], wv_eff[:, MSGW:],
         wq_eff[:, :MSGW], wq_eff[:, MSGW:]], axis=1)
    b_all = jnp.concatenate(
        [bk_eff[:MSGW], bv_eff[:MSGW],
         bk_eff[MSGW:], bv_eff[MSGW:],
         bq_eff[:MSGW], bq_eff[MSGW:]], axis=0)

    alpha = jax.nn.sigmoid(skip)
    col = jnp.arange(H * DK)
    head = 4 * (col // 64) + 2 * ((col % 64) // 32) + ((col // 8) & 1)
    dd = 2 * (col % 8) + ((col // 16) & 1)
    wa_s = (Wa * alpha)[head * DK + dd, :]
    ba_s = ba * alpha
    beta = (1.0 - alpha).astype(jnp.float32).reshape(1, 1)

    kv0, kv1, q0, q1 = _run_proj(x, w_all, b_all, bn=1000)

    ch = 100
    eps = e // NS
    npc = eps // ch // NPASS
    src3 = edge_index[0].reshape(NPASS, NS, npc, ch)
    dst3 = edge_index[1].reshape(NPASS, NS, npc, ch)
    npad = ((n + NS * 8 - 1) // (NS * 8)) * NS * 8
    zero_rows = jnp.zeros((npad // NS, ACCW), jnp.float32)
    accs = _sc_edge_kernel(npad, e, ch=ch)(kv0, kv1, q0, q1, src3, dst3,
                                           zero_rows)

    return _run_final(accs, x, wa_s, ba_s, beta, bn=1000)


# speedup vs baseline: 1.2208x; 1.0308x over previous
import functools
import math

import jax
import jax.numpy as jnp
from jax import lax
from jax.experimental import pallas as pl
from jax.experimental.pallas import tpu as pltpu
from jax.experimental.pallas import tpu_sc as plsc

H = 8
HH = 4
DK = 16
MSGW = HH * DK
ACCW = MSGW + 16

NC = 2
NS = 16
LANES = 16
NB = 2
NPASS = 2


def _proj_body(x_ref, w_ref, b_ref, kv0_ref, kv1_ref, q0_ref, q1_ref):
    xb = x_ref[...]
    y = jnp.dot(xb, w_ref[...], preferred_element_type=jnp.float32) + b_ref[...]
    yb = y.astype(jnp.bfloat16)
    kv0_ref[...] = yb[:, : 2 * MSGW]
    kv1_ref[...] = yb[:, 2 * MSGW : 4 * MSGW]
    q0_ref[...] = yb[:, 4 * MSGW : 5 * MSGW]
    q1_ref[...] = yb[:, 5 * MSGW : 6 * MSGW]


def _run_proj(x, w_all, b_all, bn):
    n, in_dim = x.shape
    grid = n // bn
    wcols = w_all.shape[1]
    return pl.pallas_call(
        _proj_body,
        grid=(grid,),
        in_specs=[
            pl.BlockSpec((bn, in_dim), lambda i: (i, 0)),
            pl.BlockSpec((in_dim, wcols), lambda i: (0, 0)),
            pl.BlockSpec((1, wcols), lambda i: (0, 0)),
        ],
        out_specs=[
            pl.BlockSpec((bn, 2 * MSGW), lambda i: (i, 0)),
            pl.BlockSpec((bn, 2 * MSGW), lambda i: (i, 0)),
            pl.BlockSpec((bn, MSGW), lambda i: (i, 0)),
            pl.BlockSpec((bn, MSGW), lambda i: (i, 0)),
        ],
        out_shape=[
            jax.ShapeDtypeStruct((n, 2 * MSGW), jnp.bfloat16),
            jax.ShapeDtypeStruct((n, 2 * MSGW), jnp.bfloat16),
            jax.ShapeDtypeStruct((n, MSGW), jnp.bfloat16),
            jax.ShapeDtypeStruct((n, MSGW), jnp.bfloat16),
        ],
    )(x, w_all, b_all.reshape(1, -1))


def _sc_edge_kernel(npad, e, ch):
    eps = e // NS
    nchunk = eps // ch
    npc = nchunk // NPASS
    rows_per_sub = npad // NS
    assert eps % ch == 0 and nchunk % NPASS == 0 and npc % NB == 0

    mesh = plsc.VectorSubcoreMesh(core_axis_name="c", subcore_axis_name="s")

    @functools.partial(
        pl.kernel,
        out_type=jax.ShapeDtypeStruct((NC, npad, ACCW), jnp.float32),
        mesh=mesh,
        compiler_params=pltpu.CompilerParams(needs_layout_passes=False,
                                             use_tc_tiling_on_sc=False),
        scratch_types=[
            pltpu.VMEM((nchunk // NPASS, ch), jnp.int32),
            pltpu.VMEM((nchunk // NPASS, ch), jnp.int32),
            pltpu.VMEM((NB, ch, 2 * MSGW), jnp.bfloat16),
            pltpu.VMEM((NB, ch, MSGW), jnp.bfloat16),
            pltpu.VMEM((NB, ch, ACCW), jnp.float32),
            pltpu.VMEM_SHARED((npad, ACCW), jnp.float32),
            pltpu.SemaphoreType.DMA,
            pltpu.SemaphoreType.DMA,
            pltpu.SemaphoreType.DMA,
            pltpu.SemaphoreType.DMA,
        ],
    )
    def edge_kernel(kv0_hbm, kv1_hbm, q0_hbm, q1_hbm, src_hbm, dst_hbm,
                    zero_hbm, out_hbm, sidx, didx, kvb, qb, ob, acc,
                    gsem0, gsem1, ssem0, ssem1):
        c = lax.axis_index("c")
        s = lax.axis_index("s")
        gsem = (gsem0, gsem1)
        ssem = (ssem0, ssem1)

        r0 = s * rows_per_sub
        pltpu.sync_copy(zero_hbm.at[pl.ds(0, rows_per_sub)],
                        acc.at[pl.ds(r0, rows_per_sub)])
        plsc.subcore_barrier()

        lane = lax.broadcasted_iota(jnp.int32, (LANES,), 0)
        x4 = lane ^ 4
        x2 = lane ^ 2
        x1 = lane ^ 1
        pidx = (lane & 1) * 8

        def _g(v, ix):
            return jnp.take_along_axis(v, ix, axis=0)

        def fire(b, j):
            @pl.when(c == 0)
            def _():
                pltpu.async_copy(kv0_hbm.at[sidx.at[j]], kvb.at[b], gsem[b])
                pltpu.async_copy(q0_hbm.at[didx.at[j]], qb.at[b], gsem[b])

            @pl.when(c == 1)
            def _():
                pltpu.async_copy(kv1_hbm.at[sidx.at[j]], kvb.at[b], gsem[b])
                pltpu.async_copy(q1_hbm.at[didx.at[j]], qb.at[b], gsem[b])

        def wait_gather(b, j):
            pltpu.make_async_copy(kv0_hbm.at[sidx.at[j]], kvb.at[b],
                                  gsem[b]).wait()
            pltpu.make_async_copy(q0_hbm.at[didx.at[j]], qb.at[b],
                                  gsem[b]).wait()

        def wait_scatter(b, j):
            pltpu.make_async_copy(ob.at[b], acc.at[didx.at[j]],
                                  ssem[b]).wait()

        def compute(b):
            kvb_b = kvb.at[b]
            qb_b = qb.at[b]
            ob_b = ob.at[b]

            unp = functools.partial(plsc.unpack,
                                    format=plsc.PackFormat.INTERLEAVED)

            @plsc.parallel_loop(0, ch, step=1, unroll=4)
            def _(i):
                ps = []
                for pr in range(HH // 2):
                    mq = (qb_b[i, pl.ds(pr * 32, 32)]
                          * kvb_b[i, pl.ds(pr * 32, 32)])
                    me, mo = unp(mq)
                    m = me + mo
                    r = m + _g(m, x4)
                    r = r + _g(r, x2)
                    r = r + _g(r, x1)
                    ps.append(jnp.exp(r))
                p01, p23 = ps
                pden = jnp.where((lane & 2) == 0, _g(p01, pidx),
                                 _g(p23, pidx))
                ob_b[i, pl.ds(MSGW, LANES)] = jnp.where(lane < HH, pden, 0.0)
                for pr in range(HH // 2):
                    ve, vo = unp(kvb_b[i, pl.ds(MSGW + pr * 32, 32)])
                    ob_b[i, pl.ds(pr * 32, DK)] = ps[pr] * ve
                    ob_b[i, pl.ds(pr * 32 + DK, DK)] = ps[pr] * vo

        def run_pass(pa, carry0):
            pltpu.sync_copy(src_hbm.at[pa, s], sidx)
            pltpu.sync_copy(dst_hbm.at[pa, s], didx)

            for b in range(min(NB, npc)):
                fire(b, b)

            def outer(i, carry):
                j0 = i * NB
                for b in range(NB):
                    j = j0 + b
                    wait_gather(b, j)

                    @pl.when(j >= NB)
                    def _():
                        wait_scatter(b, j - NB)

                    compute(b)
                    pltpu.async_copy(ob.at[b], acc.at[didx.at[j]], ssem[b],
                                     add=True)

                    @pl.when(j + NB < npc)
                    def _():
                        fire(b, j + NB)
                return carry

            lax.fori_loop(0, npc // NB, outer, 0, unroll=False)

            for b in range(NB):
                wait_scatter(b, npc - NB + b)
            return carry0

        lax.fori_loop(0, NPASS, run_pass, 0, unroll=False)

        plsc.subcore_barrier()
        pltpu.sync_copy(acc.at[pl.ds(r0, rows_per_sub)],
                        out_hbm.at[c, pl.ds(r0, rows_per_sub)])

    return edge_kernel


def _final_body(acc_ref, x_ref, wa_ref, ba_ref, beta_ref, out_ref):
    a0 = acc_ref[0]
    a1 = acc_ref[1]
    num = jnp.concatenate([a0[:, :MSGW], a1[:, :MSGW]], axis=1)
    den = jnp.concatenate([a0[:, MSGW : MSGW + HH],
                           a1[:, MSGW : MSGW + HH]], axis=1)
    rows = lax.broadcasted_iota(jnp.int32, (H, H * DK), 0)
    colv = lax.broadcasted_iota(jnp.int32, (H, H * DK), 1)
    head_col = (4 * (colv // 64) + 2 * ((colv % 64) // 32)
                + ((colv // 8) & 1))
    emat = (rows == head_col).astype(jnp.float32)
    den_full = jnp.dot(den, emat, preferred_element_type=jnp.float32)
    agg = num / (den_full + 1e-9)
    out = jnp.dot(agg, wa_ref[...], preferred_element_type=jnp.float32)
    out_ref[...] = out + ba_ref[...] + beta_ref[0, 0] * x_ref[...]


def _run_final(accs, x, wa_s, ba_s, beta, bn):
    n, in_dim = x.shape
    grid = n // bn
    return pl.pallas_call(
        _final_body,
        grid=(grid,),
        in_specs=[
            pl.BlockSpec((NC, bn, ACCW), lambda i: (0, i, 0)),
            pl.BlockSpec((bn, in_dim), lambda i: (i, 0)),
            pl.BlockSpec((H * DK, in_dim), lambda i: (0, 0)),
            pl.BlockSpec((1, in_dim), lambda i: (0, 0)),
            pl.BlockSpec(memory_space=pltpu.SMEM),
        ],
        out_specs=pl.BlockSpec((bn, in_dim), lambda i: (i, 0)),
        out_shape=jax.ShapeDtypeStruct((n, in_dim), jnp.float32),
    )(accs, x, wa_s, ba_s.reshape(1, -1), beta)


def kernel(x, edge_index, Wk, bk, Wv, bv, Wq, bq, Wa, ba, rel_att, rel_msg,
           rel_pri, skip):
    n, in_dim = x.shape
    e = edge_index.shape[1]

    wk_eff = jnp.einsum("ihd,hdk->ihk", Wk.reshape(in_dim, H, DK),
                        rel_att).reshape(in_dim, H * DK)
    bk_eff = jnp.einsum("hd,hdk->hk", bk.reshape(H, DK),
                        rel_att).reshape(H * DK)
    wv_eff = jnp.einsum("ihd,hdk->ihk", Wv.reshape(in_dim, H, DK),
                        rel_msg).reshape(in_dim, H * DK)
    bv_eff = jnp.einsum("hd,hdk->hk", bv.reshape(H, DK),
                        rel_msg).reshape(H * DK)
    scale = jnp.repeat(rel_pri, DK) / math.sqrt(DK)
    wq_eff = Wq * scale[None, :]
    bq_eff = bq * scale

    w_all = jnp.concatenate(
        [wk_eff[:, :MSGW], wv_eff[:, :MSGW],
         wk_eff[:, MSGW:], wv_eff[:, MSGW:],
         wq_eff[:, :MSGW], wq_eff[:, MSGW:]], axis=1)
    b_all = jnp.concatenate(
        [bk_eff[:MSGW], bv_eff[:MSGW],
         bk_eff[MSGW:], bv_eff[MSGW:],
         bq_eff[:MSGW], bq_eff[MSGW:]], axis=0)

    alpha = jax.nn.sigmoid(skip)
    col = jnp.arange(H * DK)
    head = 4 * (col // 64) + 2 * ((col % 64) // 32) + ((col // 8) & 1)
    dd = 2 * (col % 8) + ((col // 16) & 1)
    wa_s = (Wa * alpha)[head * DK + dd, :]
    ba_s = ba * alpha
    beta = (1.0 - alpha).astype(jnp.float32).reshape(1, 1)

    kv0, kv1, q0, q1 = _run_proj(x, w_all, b_all, bn=1000)

    ch = 125
    eps = e // NS
    npc = eps // ch // NPASS
    src3 = edge_index[0].reshape(NPASS, NS, npc, ch)
    dst3 = edge_index[1].reshape(NPASS, NS, npc, ch)
    npad = ((n + NS * 8 - 1) // (NS * 8)) * NS * 8
    zero_rows = jnp.zeros((npad // NS, ACCW), jnp.float32)
    accs = _sc_edge_kernel(npad, e, ch=ch)(kv0, kv1, q0, q1, src3, dst3,
                                           zero_rows)

    return _run_final(accs, x, wa_s, ba_s, beta, bn=1000)
